# baseline (speedup 1.0000x reference)
import functools

import jax
import jax.numpy as jnp
from jax import lax
from jax.experimental import pallas as pl
from jax.experimental.pallas import tpu as pltpu
from jax.experimental.pallas import tpu_sc as plsc

_NC = 2
_NS = 16
_NW = _NC * _NS
_CH = 128
_KB = 8
_BR = 1000


def _cdiv(a, b):
    return (a + b - 1) // b


def _sc_segsum_rows(xl, src3, dst3, zrows, zdeg, ones, n_pad, rpt, nch, d):
    mesh = plsc.VectorSubcoreMesh(core_axis_name="c", subcore_axis_name="s",
                                  num_cores=_NC, num_subcores=_NS)
    nblk = nch // _KB

    @functools.partial(
        pl.kernel,
        out_type=[jax.ShapeDtypeStruct((_NC, n_pad, d), jnp.float32),
                  jax.ShapeDtypeStruct((_NC, n_pad), jnp.float32)],
        mesh=mesh,
        scratch_types=[
            pltpu.VMEM((_KB, _CH), jnp.int32),
            pltpu.VMEM((_KB, _CH), jnp.int32),
            pltpu.VMEM((_KB, _CH), jnp.int32),
            pltpu.VMEM((_KB, _CH), jnp.int32),
            pltpu.VMEM((_CH, d), jnp.float32),
            pltpu.VMEM((_CH, d), jnp.float32),
            pltpu.VMEM((_CH,), jnp.float32),
            pltpu.VMEM((rpt,), jnp.float32),
            pltpu.VMEM_SHARED((n_pad, d), jnp.float32),
            pltpu.VMEM_SHARED((n_pad,), jnp.float32),
            pltpu.SemaphoreType.DMA,
            pltpu.SemaphoreType.DMA,
            pltpu.SemaphoreType.DMA,
            pltpu.SemaphoreType.DMA,
            pltpu.SemaphoreType.DMA,
            pltpu.SemaphoreType.DMA,
            pltpu.SemaphoreType.DMA,
            pltpu.SemaphoreType.DMA,
        ],
    )
    def k(xl_hbm, src_hbm, dst_hbm, zrows_hbm, zdeg_hbm, ones_hbm,
          acc_out, deg_out, srci0, srci1, dsti0, dsti1, rows0_v, rows1_v,
          ones_v, degb_v, acc_sh, deg_sh, i0, i1, g0, g1, s0, s1, d0, d1):
        srci = (srci0, srci1)
        dsti = (dsti0, dsti1)
        rows = (rows0_v, rows1_v)
        isem = (i0, i1)
        gsem = (g0, g1)
        ssem = (s0, s1)
        dsem = (d0, d1)
        c = lax.axis_index("c")
        s = lax.axis_index("s")
        w = c * _NS + s
        pltpu.sync_copy(zrows_hbm, rows0_v)
        for t in range(rpt // _CH):
            pltpu.sync_copy(rows0_v, acc_sh.at[pl.ds(s * rpt + t * _CH, _CH)])
        pltpu.sync_copy(zdeg_hbm, degb_v)
        pltpu.sync_copy(degb_v, deg_sh.at[pl.ds(s * rpt, rpt)])
        pltpu.sync_copy(ones_hbm, ones_v)
        for b in range(2):
            pltpu.async_copy(src_hbm.at[w, pl.ds(b * _KB, _KB)], srci[b],
                             isem[b])
            pltpu.async_copy(dst_hbm.at[w, pl.ds(b * _KB, _KB)], dsti[b],
                             isem[b])
        plsc.subcore_barrier()

        def blockpair(it, carry):
            for b in range(2):
                bk = it * 2 + b
                pltpu.make_async_copy(
                    src_hbm.at[w, pl.ds(bk * _KB, _KB)], srci[b],
                    isem[b]).wait()
                pltpu.make_async_copy(
                    dst_hbm.at[w, pl.ds(bk * _KB, _KB)], dsti[b],
                    isem[b]).wait()
                for rb in range(2):
                    pltpu.async_copy(xl_hbm.at[srci[b].at[rb]], rows[rb],
                                     gsem[rb])
                for j in range(_KB):
                    rb = j % 2
                    pltpu.async_copy(ones_v, deg_sh.at[dsti[b].at[j]],
                                     dsem[rb], add=True)
                    pltpu.make_async_copy(
                        xl_hbm.at[srci[b].at[j]], rows[rb], gsem[rb]).wait()
                    pltpu.async_copy(rows[rb], acc_sh.at[dsti[b].at[j]],
                                     ssem[rb], add=True)
                    pltpu.make_async_copy(
                        rows[rb], acc_sh.at[dsti[b].at[j]], ssem[rb]).wait()
                    if j + 2 < _KB:
                        pltpu.async_copy(xl_hbm.at[srci[b].at[j + 2]],
                                         rows[rb], gsem[rb])
                    pltpu.make_async_copy(
                        ones_v, deg_sh.at[dsti[b].at[j]], dsem[rb]).wait()
                pltpu.async_copy(src_hbm.at[w, pl.ds((bk + 2) * _KB, _KB)],
                                 srci[b], isem[b])
                pltpu.async_copy(dst_hbm.at[w, pl.ds((bk + 2) * _KB, _KB)],
                                 dsti[b], isem[b])
            return carry

        lax.fori_loop(0, nblk // 2, blockpair, 0)
        for b in range(2):
            pltpu.make_async_copy(
                src_hbm.at[w, pl.ds(b * _KB, _KB)], srci[b], isem[b]).wait()
            pltpu.make_async_copy(
                dst_hbm.at[w, pl.ds(b * _KB, _KB)], dsti[b], isem[b]).wait()
        plsc.subcore_barrier()
        for t in range(rpt // _CH):
            pltpu.sync_copy(acc_sh.at[pl.ds(s * rpt + t * _CH, _CH)], rows0_v)
            pltpu.sync_copy(rows0_v,
                            acc_out.at[c, pl.ds(s * rpt + t * _CH, _CH)])
        pltpu.sync_copy(deg_sh.at[pl.ds(s * rpt, rpt)], degb_v)
        pltpu.sync_copy(degb_v, deg_out.at[c, pl.ds(s * rpt, rpt)])

    return k(xl, src3, dst3, zrows, zdeg, ones)


def _sc_segsum_scalar(z, src3, dst3, zdeg, n_pad, rpt, nch):
    mesh = plsc.VectorSubcoreMesh(core_axis_name="c", subcore_axis_name="s",
                                  num_cores=_NC, num_subcores=_NS)
    nrow = nch + 2 * _KB

    @functools.partial(
        pl.kernel,
        out_type=jax.ShapeDtypeStruct((_NC, n_pad), jnp.float32),
        mesh=mesh,
        scratch_types=[
            pltpu.VMEM((nrow, _CH), jnp.int32),
            pltpu.VMEM((nrow, _CH), jnp.int32),
            pltpu.VMEM((_CH,), jnp.float32),
            pltpu.VMEM((_CH,), jnp.float32),
            pltpu.VMEM((rpt,), jnp.float32),
            pltpu.VMEM_SHARED((n_pad,), jnp.float32),
            pltpu.SemaphoreType.DMA,
            pltpu.SemaphoreType.DMA,
            pltpu.SemaphoreType.DMA,
            pltpu.SemaphoreType.DMA,
        ],
    )
    def k(z_hbm, src_hbm, dst_hbm, zdeg_hbm,
          agg_out, src_v, dst_v, val0_v, val1_v, zb_v, acc_sh,
          g0, g1, s0, s1):
        vals = (val0_v, val1_v)
        gsem = (g0, g1)
        ssem = (s0, s1)
        c = lax.axis_index("c")
        s = lax.axis_index("s")
        pltpu.sync_copy(zdeg_hbm, zb_v)
        pltpu.sync_copy(zb_v, acc_sh.at[pl.ds(s * rpt, rpt)])
        pltpu.sync_copy(src_hbm.at[c * _NS + s], src_v)
        pltpu.sync_copy(dst_hbm.at[c * _NS + s], dst_v)
        for b in range(2):
            pltpu.async_copy(z_hbm.at[src_v.at[b]], vals[b], gsem[b])
        plsc.subcore_barrier()

        def pair(jh, carry):
            j = jh * 2
            for b in range(2):
                jj = j + b
                pltpu.make_async_copy(
                    z_hbm.at[src_v.at[jj]], vals[b], gsem[b]).wait()
                pltpu.async_copy(vals[b], acc_sh.at[dst_v.at[jj]], ssem[b],
                                 add=True)
                pltpu.make_async_copy(
                    vals[b], acc_sh.at[dst_v.at[jj]], ssem[b]).wait()
                pltpu.async_copy(z_hbm.at[src_v.at[jj + 2]], vals[b],
                                 gsem[b])
            return carry

        lax.fori_loop(0, nch // 2, pair, 0)
        for b in range(2):
            pltpu.make_async_copy(
                z_hbm.at[src_v.at[b]], vals[b], gsem[b]).wait()
        plsc.subcore_barrier()
        pltpu.sync_copy(acc_sh.at[pl.ds(s * rpt, rpt)], zb_v)
        pltpu.sync_copy(zb_v, agg_out.at[c, pl.ds(s * rpt, rpt)])

    return k(z, src3, dst3, zdeg)


def _tc_lin1(x, wlr):
    n, d = x.shape
    g = n // _BR

    def body(x_ref, w_ref, xl_ref, xr_ref):
        xb = x_ref[...]
        xl_ref[...] = jnp.dot(xb, w_ref[0], preferred_element_type=jnp.float32)
        xr_ref[...] = jnp.dot(xb, w_ref[1], preferred_element_type=jnp.float32)

    return pl.pallas_call(
        body,
        grid=(g,),
        in_specs=[pl.BlockSpec((_BR, d), lambda i: (i, 0)),
                  pl.BlockSpec((2, d, d), lambda i: (0, 0, 0))],
        out_specs=[pl.BlockSpec((_BR, d), lambda i: (i, 0)),
                   pl.BlockSpec((_BR, d), lambda i: (i, 0))],
        out_shape=[jax.ShapeDtypeStruct((n, d), jnp.float32),
                   jax.ShapeDtypeStruct((n, d), jnp.float32)],
    )(x, wlr)


def _tc_mid(accp, degp, xr, b1r, w2p, n):
    d = xr.shape[1]
    g = n // _BR

    def body(acc_ref, deg_ref, xr_ref, b1_ref, w2_ref, zhr_ref, degs_ref):
        aggsum = acc_ref[0] + acc_ref[1]
        deg = jnp.maximum(deg_ref[0] + deg_ref[1], 1.0)
        h = jnp.maximum(aggsum / deg + b1_ref[...] + xr_ref[...], 0.0)
        zhr_ref[...] = jnp.dot(h, w2_ref[...], preferred_element_type=jnp.float32)
        degs_ref[...] = deg

    return pl.pallas_call(
        body,
        grid=(g,),
        in_specs=[pl.BlockSpec((_NC, _BR, d), lambda i: (0, i, 0)),
                  pl.BlockSpec((_NC, _BR, 1), lambda i: (0, i, 0)),
                  pl.BlockSpec((_BR, d), lambda i: (i, 0)),
                  pl.BlockSpec((1, d), lambda i: (0, 0)),
                  pl.BlockSpec((d, 8), lambda i: (0, 0))],
        out_specs=[pl.BlockSpec((_BR, 8), lambda i: (i, 0)),
                   pl.BlockSpec((_BR, 1), lambda i: (i, 0))],
        out_shape=[jax.ShapeDtypeStruct((n, 8), jnp.float32),
                   jax.ShapeDtypeStruct((n, 1), jnp.float32)],
    )(accp, degp, xr, b1r, w2p)


def _tc_out(aggzp, degs, zhr, b2r, n):
    g = n // _BR

    def body(aggz_ref, deg_ref, zhr_ref, b2_ref, out_ref, sig_ref):
        aggz = aggz_ref[0] + aggz_ref[1]
        hr = zhr_ref[:, 1:2]
        o = aggz / deg_ref[...] + b2_ref[0, 0] + hr
        out_ref[...] = o
        sig_ref[...] = jax.nn.sigmoid(o)

    return pl.pallas_call(
        body,
        grid=(g,),
        in_specs=[pl.BlockSpec((_NC, _BR, 1), lambda i: (0, i, 0)),
                  pl.BlockSpec((_BR, 1), lambda i: (i, 0)),
                  pl.BlockSpec((_BR, 8), lambda i: (i, 0)),
                  pl.BlockSpec((1, 1), lambda i: (0, 0))],
        out_specs=[pl.BlockSpec((_BR, 1), lambda i: (i, 0)),
                   pl.BlockSpec((_BR, 1), lambda i: (i, 0))],
        out_shape=[jax.ShapeDtypeStruct((n, 1), jnp.float32),
                   jax.ShapeDtypeStruct((n, 1), jnp.float32)],
    )(aggzp, degs, zhr, b2r)


def kernel(x, edge_index, W1_l, b1, W1_r, W2_l, b2, W2_r):
    n, d = x.shape
    e = edge_index.shape[1]
    nch = _KB * _cdiv(e, _NW * _CH * _KB)
    ept = nch * _CH
    e_pad = ept * _NW
    rpt = _CH * _cdiv(n + 1, _NS * _CH)
    n_pad = rpt * _NS

    pad = e_pad - e
    src3 = jnp.concatenate(
        [edge_index[0], jnp.zeros((pad,), jnp.int32)]).reshape(_NW, nch, _CH)
    dst3 = jnp.concatenate(
        [edge_index[1], jnp.full((pad,), n, jnp.int32)]).reshape(_NW, nch, _CH)
    src3 = jnp.concatenate(
        [src3, jnp.zeros((_NW, 2 * _KB, _CH), jnp.int32)], axis=1)
    dst3 = jnp.concatenate(
        [dst3, jnp.full((_NW, 2 * _KB, _CH), n, jnp.int32)], axis=1)
    zrows = jnp.zeros((_CH, d), jnp.float32)
    zdeg = jnp.zeros((rpt,), jnp.float32)
    ones = jnp.ones((_CH,), jnp.float32)

    xl, xr = _tc_lin1(x, jnp.stack([W1_l, W1_r]))
    accp, degp = _sc_segsum_rows(xl, src3, dst3, zrows, zdeg, ones,
                                 n_pad, rpt, nch, d)
    degp3 = degp.reshape(_NC, n_pad, 1)
    b1r = b1.reshape(1, d)
    w2p = jnp.zeros((d, 8), jnp.float32)
    w2p = w2p.at[:, 0].set(W2_l[:, 0]).at[:, 1].set(W2_r[:, 0])
    zhr, degs = _tc_mid(accp, degp3, xr, b1r, w2p, n)
    z = jnp.concatenate([zhr[:, 0], jnp.zeros((n_pad - n,), jnp.float32)])
    aggzp = _sc_segsum_scalar(z, src3, dst3, zdeg, n_pad, rpt, nch)
    aggzp = aggzp.reshape(_NC, n_pad, 1)
    b2r = b2.reshape(1, 1)
    out, sig = _tc_out(aggzp, degs, zhr, b2r, n)
    return (out, sig)


# speedup vs baseline: 2.0954x; 2.0954x over previous
import functools

import jax
import jax.numpy as jnp
from jax import lax
from jax.experimental import pallas as pl
from jax.experimental.pallas import tpu as pltpu
from jax.experimental.pallas import tpu_sc as plsc

_NC = 2
_NS = 16
_NW = _NC * _NS
_CH = 128
_KB = 8
_BR = 1000


def _cdiv(a, b):
    return (a + b - 1) // b


def _sc_segsum_rows(xl, src3, dst3, zrows, zdeg, ones, n_pad, rpt, nch, d):
    mesh = plsc.VectorSubcoreMesh(core_axis_name="c", subcore_axis_name="s",
                                  num_cores=_NC, num_subcores=_NS)
    nblk = nch // _KB

    @functools.partial(
        pl.kernel,
        out_type=[jax.ShapeDtypeStruct((_NC, n_pad, d), jnp.float32),
                  jax.ShapeDtypeStruct((_NC, n_pad), jnp.float32)],
        mesh=mesh,
        scratch_types=[
            pltpu.VMEM((_KB, _CH), jnp.int32),
            pltpu.VMEM((_KB, _CH), jnp.int32),
            pltpu.VMEM((_KB, _CH), jnp.int32),
            pltpu.VMEM((_KB, _CH), jnp.int32),
            pltpu.VMEM((_CH, d), jnp.float32),
            pltpu.VMEM((_CH, d), jnp.float32),
            pltpu.VMEM((_CH,), jnp.float32),
            pltpu.VMEM((rpt,), jnp.float32),
            pltpu.VMEM_SHARED((n_pad, d), jnp.float32),
            pltpu.VMEM_SHARED((n_pad,), jnp.float32),
            pltpu.SemaphoreType.DMA,
            pltpu.SemaphoreType.DMA,
            pltpu.SemaphoreType.DMA,
            pltpu.SemaphoreType.DMA,
            pltpu.SemaphoreType.DMA,
            pltpu.SemaphoreType.DMA,
            pltpu.SemaphoreType.DMA,
            pltpu.SemaphoreType.DMA,
        ],
    )
    def k(xl_hbm, src_hbm, dst_hbm, zrows_hbm, zdeg_hbm, ones_hbm,
          acc_out, deg_out, srci0, srci1, dsti0, dsti1, rows0_v, rows1_v,
          ones_v, degb_v, acc_sh, deg_sh, i0, i1, g0, g1, s0, s1, d0, d1):
        srci = (srci0, srci1)
        dsti = (dsti0, dsti1)
        rows = (rows0_v, rows1_v)
        isem = (i0, i1)
        gsem = (g0, g1)
        ssem = (s0, s1)
        dsem = (d0, d1)
        c = lax.axis_index("c")
        s = lax.axis_index("s")
        w = c * _NS + s
        pltpu.sync_copy(zrows_hbm, rows0_v)
        for t in range(rpt // _CH):
            pltpu.sync_copy(rows0_v, acc_sh.at[pl.ds(s * rpt + t * _CH, _CH)])
        pltpu.sync_copy(zdeg_hbm, degb_v)
        pltpu.sync_copy(degb_v, deg_sh.at[pl.ds(s * rpt, rpt)])
        pltpu.sync_copy(ones_hbm, ones_v)
        for b in range(2):
            pltpu.async_copy(src_hbm.at[w, pl.ds(b * _KB, _KB)], srci[b],
                             isem[b])
            pltpu.async_copy(dst_hbm.at[w, pl.ds(b * _KB, _KB)], dsti[b],
                             isem[b])
        plsc.subcore_barrier()

        def blockpair(it, carry):
            for b in range(2):
                bk = it * 2 + b
                pltpu.make_async_copy(
                    src_hbm.at[w, pl.ds(bk * _KB, _KB)], srci[b],
                    isem[b]).wait()
                pltpu.make_async_copy(
                    dst_hbm.at[w, pl.ds(bk * _KB, _KB)], dsti[b],
                    isem[b]).wait()
                for rb in range(2):
                    pltpu.async_copy(xl_hbm.at[srci[b].at[rb]], rows[rb],
                                     gsem[rb])
                for j in range(_KB):
                    rb = j % 2
                    pltpu.async_copy(ones_v, deg_sh.at[dsti[b].at[j]],
                                     dsem[rb], add=True)
                    pltpu.make_async_copy(
                        xl_hbm.at[srci[b].at[j]], rows[rb], gsem[rb]).wait()
                    pltpu.async_copy(rows[rb], acc_sh.at[dsti[b].at[j]],
                                     ssem[rb], add=True)
                    pltpu.make_async_copy(
                        rows[rb], acc_sh.at[dsti[b].at[j]], ssem[rb]).wait()
                    if j + 2 < _KB:
                        pltpu.async_copy(xl_hbm.at[srci[b].at[j + 2]],
                                         rows[rb], gsem[rb])
                    pltpu.make_async_copy(
                        ones_v, deg_sh.at[dsti[b].at[j]], dsem[rb]).wait()
                pltpu.async_copy(src_hbm.at[w, pl.ds((bk + 2) * _KB, _KB)],
                                 srci[b], isem[b])
                pltpu.async_copy(dst_hbm.at[w, pl.ds((bk + 2) * _KB, _KB)],
                                 dsti[b], isem[b])
            return carry

        lax.fori_loop(0, nblk // 2, blockpair, 0)
        for b in range(2):
            pltpu.make_async_copy(
                src_hbm.at[w, pl.ds(b * _KB, _KB)], srci[b], isem[b]).wait()
            pltpu.make_async_copy(
                dst_hbm.at[w, pl.ds(b * _KB, _KB)], dsti[b], isem[b]).wait()
        plsc.subcore_barrier()
        for t in range(rpt // _CH):
            pltpu.sync_copy(acc_sh.at[pl.ds(s * rpt + t * _CH, _CH)], rows0_v)
            pltpu.sync_copy(rows0_v,
                            acc_out.at[c, pl.ds(s * rpt + t * _CH, _CH)])
        pltpu.sync_copy(deg_sh.at[pl.ds(s * rpt, rpt)], degb_v)
        pltpu.sync_copy(degb_v, deg_out.at[c, pl.ds(s * rpt, rpt)])

    return k(xl, src3, dst3, zrows, zdeg, ones)


def _sc_segsum_scalar(z, src3, dst3, zdeg, n_pad, rpt, nch):
    mesh = plsc.VectorSubcoreMesh(core_axis_name="c", subcore_axis_name="s",
                                  num_cores=_NC, num_subcores=_NS)
    nrow = nch + 2 * _KB

    @functools.partial(
        pl.kernel,
        out_type=jax.ShapeDtypeStruct((_NC, n_pad), jnp.float32),
        mesh=mesh,
        scratch_types=[
            pltpu.VMEM((nrow, _CH), jnp.int32),
            pltpu.VMEM((nrow, _CH), jnp.int32),
            pltpu.VMEM((_CH,), jnp.float32),
            pltpu.VMEM((_CH,), jnp.float32),
            pltpu.VMEM((rpt,), jnp.float32),
            pltpu.VMEM_SHARED((n_pad,), jnp.float32),
            pltpu.SemaphoreType.DMA,
            pltpu.SemaphoreType.DMA,
            pltpu.SemaphoreType.DMA,
            pltpu.SemaphoreType.DMA,
        ],
    )
    def k(z_hbm, src_hbm, dst_hbm, zdeg_hbm,
          agg_out, src_v, dst_v, val0_v, val1_v, zb_v, acc_sh,
          g0, g1, s0, s1):
        vals = (val0_v, val1_v)
        gsem = (g0, g1)
        ssem = (s0, s1)
        c = lax.axis_index("c")
        s = lax.axis_index("s")
        pltpu.sync_copy(zdeg_hbm, zb_v)
        pltpu.sync_copy(zb_v, acc_sh.at[pl.ds(s * rpt, rpt)])
        pltpu.sync_copy(src_hbm.at[c * _NS + s], src_v)
        pltpu.sync_copy(dst_hbm.at[c * _NS + s], dst_v)
        for b in range(2):
            pltpu.async_copy(z_hbm.at[src_v.at[b]], vals[b], gsem[b])
        plsc.subcore_barrier()

        def pair(jh, carry):
            j = jh * 2
            for b in range(2):
                jj = j + b
                pltpu.make_async_copy(
                    z_hbm.at[src_v.at[jj]], vals[b], gsem[b]).wait()
                pltpu.async_copy(vals[b], acc_sh.at[dst_v.at[jj]], ssem[b],
                                 add=True)
                pltpu.make_async_copy(
                    vals[b], acc_sh.at[dst_v.at[jj]], ssem[b]).wait()
                pltpu.async_copy(z_hbm.at[src_v.at[jj + 2]], vals[b],
                                 gsem[b])
            return carry

        lax.fori_loop(0, nch // 2, pair, 0)
        for b in range(2):
            pltpu.make_async_copy(
                z_hbm.at[src_v.at[b]], vals[b], gsem[b]).wait()
        plsc.subcore_barrier()
        pltpu.sync_copy(acc_sh.at[pl.ds(s * rpt, rpt)], zb_v)
        pltpu.sync_copy(zb_v, agg_out.at[c, pl.ds(s * rpt, rpt)])

    return k(z, src3, dst3, zdeg)


def _tc_lin1(x, wlr):
    n, d = x.shape
    g = n // _BR

    def body(x_ref, w_ref, xl_ref, xr_ref):
        xb = x_ref[...]
        xl_ref[...] = jnp.dot(xb, w_ref[0], preferred_element_type=jnp.float32)
        xr_ref[...] = jnp.dot(xb, w_ref[1], preferred_element_type=jnp.float32)

    return pl.pallas_call(
        body,
        grid=(g,),
        in_specs=[pl.BlockSpec((_BR, d), lambda i: (i, 0)),
                  pl.BlockSpec((2, d, d), lambda i: (0, 0, 0))],
        out_specs=[pl.BlockSpec((_BR, d), lambda i: (i, 0)),
                   pl.BlockSpec((_BR, d), lambda i: (i, 0))],
        out_shape=[jax.ShapeDtypeStruct((n, d), jnp.float32),
                   jax.ShapeDtypeStruct((n, d), jnp.float32)],
    )(x, wlr)


def _tc_mid(accp, degp, xr, b1r, w2p, n):
    d = xr.shape[1]
    g = n // _BR

    def body(acc_ref, deg_ref, xr_ref, b1_ref, w2_ref, zhr_ref, degs_ref):
        aggsum = acc_ref[0] + acc_ref[1]
        deg = jnp.maximum(deg_ref[0] + deg_ref[1], 1.0)
        h = jnp.maximum(aggsum / deg + b1_ref[...] + xr_ref[...], 0.0)
        zhr_ref[...] = jnp.dot(h, w2_ref[...], preferred_element_type=jnp.float32)
        degs_ref[...] = deg

    return pl.pallas_call(
        body,
        grid=(g,),
        in_specs=[pl.BlockSpec((_NC, _BR, d), lambda i: (0, i, 0)),
                  pl.BlockSpec((_NC, _BR, 1), lambda i: (0, i, 0)),
                  pl.BlockSpec((_BR, d), lambda i: (i, 0)),
                  pl.BlockSpec((1, d), lambda i: (0, 0)),
                  pl.BlockSpec((d, 8), lambda i: (0, 0))],
        out_specs=[pl.BlockSpec((_BR, 8), lambda i: (i, 0)),
                   pl.BlockSpec((_BR, 1), lambda i: (i, 0))],
        out_shape=[jax.ShapeDtypeStruct((n, 8), jnp.float32),
                   jax.ShapeDtypeStruct((n, 1), jnp.float32)],
    )(accp, degp, xr, b1r, w2p)


def _tc_out(aggzp, degs, zhr, b2r, n):
    g = n // _BR

    def body(aggz_ref, deg_ref, zhr_ref, b2_ref, out_ref, sig_ref):
        aggz = aggz_ref[0] + aggz_ref[1]
        hr = zhr_ref[:, 1:2]
        o = aggz / deg_ref[...] + b2_ref[0, 0] + hr
        out_ref[...] = o
        sig_ref[...] = jax.nn.sigmoid(o)

    return pl.pallas_call(
        body,
        grid=(g,),
        in_specs=[pl.BlockSpec((_NC, _BR, 1), lambda i: (0, i, 0)),
                  pl.BlockSpec((_BR, 1), lambda i: (i, 0)),
                  pl.BlockSpec((_BR, 8), lambda i: (i, 0)),
                  pl.BlockSpec((1, 1), lambda i: (0, 0))],
        out_specs=[pl.BlockSpec((_BR, 1), lambda i: (i, 0)),
                   pl.BlockSpec((_BR, 1), lambda i: (i, 0))],
        out_shape=[jax.ShapeDtypeStruct((n, 1), jnp.float32),
                   jax.ShapeDtypeStruct((n, 1), jnp.float32)],
    )(aggzp, degs, zhr, b2r)


def kernel(x, edge_index, W1_l, b1, W1_r, W2_l, b2, W2_r):
    n, d = x.shape
    e = edge_index.shape[1]
    nch = _KB * _cdiv(e, _NW * _CH * _KB)
    ept = nch * _CH
    e_pad = ept * _NW
    rpt = _CH * _cdiv(n + 1, _NS * _CH)
    n_pad = rpt * _NS

    pad = e_pad - e
    pidx = jnp.arange(pad, dtype=jnp.int32)
    src_pad = pidx % jnp.int32(n)
    dst_pad = jnp.int32(n) + pidx % jnp.int32(n_pad - n)
    src3 = jnp.concatenate(
        [edge_index[0], src_pad]).reshape(_NW, nch, _CH)
    dst3 = jnp.concatenate(
        [edge_index[1], dst_pad]).reshape(_NW, nch, _CH)
    src3 = jnp.concatenate(
        [src3, jnp.zeros((_NW, 2 * _KB, _CH), jnp.int32)], axis=1)
    dst3 = jnp.concatenate(
        [dst3, jnp.full((_NW, 2 * _KB, _CH), n, jnp.int32)], axis=1)
    zrows = jnp.zeros((_CH, d), jnp.float32)
    zdeg = jnp.zeros((rpt,), jnp.float32)
    ones = jnp.ones((_CH,), jnp.float32)

    xl, xr = _tc_lin1(x, jnp.stack([W1_l, W1_r]))
    accp, degp = _sc_segsum_rows(xl, src3, dst3, zrows, zdeg, ones,
                                 n_pad, rpt, nch, d)
    degp3 = degp.reshape(_NC, n_pad, 1)
    b1r = b1.reshape(1, d)
    w2p = jnp.zeros((d, 8), jnp.float32)
    w2p = w2p.at[:, 0].set(W2_l[:, 0]).at[:, 1].set(W2_r[:, 0])
    zhr, degs = _tc_mid(accp, degp3, xr, b1r, w2p, n)
    z = jnp.concatenate([zhr[:, 0], jnp.zeros((n_pad - n,), jnp.float32)])
    aggzp = _sc_segsum_scalar(z, src3, dst3, zdeg, n_pad, rpt, nch)
    aggzp = aggzp.reshape(_NC, n_pad, 1)
    b2r = b2.reshape(1, 1)
    out, sig = _tc_out(aggzp, degs, zhr, b2r, n)
    return (out, sig)
